# baseline (speedup 1.0000x reference)
import functools

import jax
import jax.numpy as jnp
from jax import lax
from jax.experimental import pallas as pl
from jax.experimental.pallas import tpu as pltpu
from jax.experimental.pallas import tpu_sc as plsc

N = 50000
R = 8
E = 800000
EMB = 128
H = 16
C = 32

NC = 2
NS = 16
NW = NC * NS
CHUNK = 128
SUBS = 5
SUP = CHUNK * SUBS
NSUP = E // SUP
ITERS = (NSUP + NW - 1) // NW
NPAD = 50048


@functools.lru_cache(maxsize=None)
def _mesh():
    return plsc.VectorSubcoreMesh(core_axis_name="c", subcore_axis_name="s",
                                  num_cores=NC, num_subcores=NS)


def _wid():
    return lax.axis_index("s") * NC + lax.axis_index("c")


@functools.lru_cache(maxsize=None)
def _sc_hist():
    return pl.kernel(
        _sc_hist_body,
        compiler_params=pltpu.CompilerParams(use_tc_tiling_on_sc=False),
        out_type=jax.ShapeDtypeStruct((NC * N * R,), jnp.float32),
        mesh=_mesh(),
        scratch_types=[
            pltpu.VMEM((SUP,), jnp.int32),
            pltpu.VMEM((SUP,), jnp.int32),
            pltpu.VMEM((SUBS, CHUNK), jnp.int32),
            pltpu.VMEM((SUP,), jnp.float32),
            pltpu.VMEM(((N * R) // NS,), jnp.float32),
            pltpu.VMEM_SHARED((N * R,), jnp.float32),
            pltpu.SemaphoreType.DMA,
            pltpu.SemaphoreType.DMA,
        ],
    )


def _sc_hist_body(src_hbm, rel_hbm, zeros_hbm, out_hbm, srcv, relv, ver2d,
                  onesv, bounce, acc, semi, sems):
    c = lax.axis_index("c")
    s = lax.axis_index("s")
    wid = _wid()
    words = (N * R) // NS
    pltpu.sync_copy(zeros_hbm.at[pl.ds(s * words, words)], bounce)
    pltpu.sync_copy(bounce, acc.at[pl.ds(s * words, words)])
    ones16 = jnp.full((16,), 1.0, dtype=jnp.float32)
    for j in range(SUP // 16):
        onesv[pl.ds(j * 16, 16)] = ones16
    plsc.subcore_barrier()

    def body(g, _):
        cid = g * NW + wid

        @pl.when(cid < NSUP)
        def _():
            base = cid * SUP
            dls = [pltpu.async_copy(src_hbm.at[pl.ds(base, SUP)], srcv, semi),
                   pltpu.async_copy(rel_hbm.at[pl.ds(base, SUP)], relv, semi)]
            for d in dls:
                d.wait()
            for q in range(SUP // 16):
                sl = pl.ds(q * 16, 16)
                ver2d[q // 8, pl.ds((q % 8) * 16, 16)] = srcv[sl] * R + relv[sl]
            sds = [pltpu.async_copy(onesv.at[pl.ds(i * CHUNK, CHUNK)],
                                    acc.at[ver2d.at[i]], sems, add=True)
                   for i in range(SUBS)]
            for d in sds:
                d.wait()

        return ()

    lax.fori_loop(0, ITERS, body, (), unroll=False)
    plsc.subcore_barrier()
    pltpu.sync_copy(acc.at[pl.ds(s * words, words)], bounce)
    pltpu.sync_copy(bounce, out_hbm.at[pl.ds(c * (N * R) + s * words, words)])


_NB = ITERS // 2


def _edge_bufs():
    return [
        pltpu.VMEM((SUP,), jnp.int32),
        pltpu.VMEM((SUP,), jnp.int32),
        pltpu.VMEM((SUP,), jnp.int32),
        pltpu.VMEM((SUP,), jnp.int32),
        pltpu.VMEM((SUP,), jnp.int32),
        pltpu.VMEM((SUBS, CHUNK), jnp.int32),
        pltpu.VMEM((SUP,), jnp.float32),
        pltpu.VMEM((SUP, H), jnp.float32),
    ]


@functools.lru_cache(maxsize=None)
def _sc_edge16():
    return pl.kernel(
        _sc_edge16_body,
        compiler_params=pltpu.CompilerParams(use_tc_tiling_on_sc=False),
        out_type=(
            jax.ShapeDtypeStruct((E,), jnp.float32),
            jax.ShapeDtypeStruct((NC, NPAD, H), jnp.float32),
        ),
        mesh=_mesh(),
        scratch_types=(
            _edge_bufs() + _edge_bufs() + [
                pltpu.VMEM((NPAD // NS, H), jnp.float32),
                pltpu.VMEM_SHARED((NPAD, H), jnp.float32),
            ] + [pltpu.SemaphoreType.DMA] * 9
        ),
    )


def _sc_edge16_body(src_hbm, dst_hbm, rel_hbm, xw_hbm, invdeg_hbm, zeros_hbm,
                    vals_hbm, out_hbm,
                    srcv0, dstv0, relv0, horv0, verv0, src2d0, valsv0, rows0,
                    srcv1, dstv1, relv1, horv1, verv1, src2d1, valsv1, rows1,
                    bounce, acc,
                    semi0, semi1, semg0, semg1, semv0, semv1, sems0, sems1,
                    semw):
    c = lax.axis_index("c")
    s = lax.axis_index("s")
    wid = _wid()
    rows_per_tile = NPAD // NS
    pltpu.sync_copy(zeros_hbm.at[pl.ds(s * rows_per_tile, rows_per_tile), :],
                    bounce)
    pltpu.sync_copy(bounce, acc.at[pl.ds(s * rows_per_tile, rows_per_tile), :])
    plsc.subcore_barrier()

    P0 = (srcv0, dstv0, relv0, horv0, verv0, src2d0, valsv0, rows0,
          semi0, semg0, semv0, sems0)
    P1 = (srcv1, dstv1, relv1, horv1, verv1, src2d1, valsv1, rows1,
          semi1, semg1, semv1, sems1)

    def chunk_of(j):
        cidr = j * NW + wid
        cid = jnp.minimum(cidr, NSUP - 1)
        return cid * SUP, jnp.where(cidr < NSUP, 1.0, 0.0).astype(jnp.float32)

    def fire_idx(base, bufs):
        (srcv, dstv, relv, _, _, _, _, _, semi, _, _, _) = bufs
        pltpu.async_copy(src_hbm.at[pl.ds(base, SUP)], srcv, semi)
        pltpu.async_copy(dst_hbm.at[pl.ds(base, SUP)], dstv, semi)
        pltpu.async_copy(rel_hbm.at[pl.ds(base, SUP)], relv, semi)

    def wait_idx(base, bufs):
        (srcv, dstv, relv, _, _, _, _, _, semi, _, _, _) = bufs
        pltpu.make_async_copy(src_hbm.at[pl.ds(base, SUP)], srcv, semi).wait()
        pltpu.make_async_copy(dst_hbm.at[pl.ds(base, SUP)], dstv, semi).wait()
        pltpu.make_async_copy(rel_hbm.at[pl.ds(base, SUP)], relv, semi).wait()

    def compute_and_fire(bufs):
        (srcv, dstv, relv, horv, verv, src2d, valsv, rows,
         _, semg, semv, _) = bufs
        for q in range(SUP // 16):
            sl = pl.ds(q * 16, 16)
            rj = relv[sl]
            sj = srcv[sl]
            horv[sl] = dstv[sl] * R + rj
            verv[sl] = sj * R + rj
            src2d[q // 8, pl.ds((q % 8) * 16, 16)] = sj
        gds = [pltpu.async_copy(
                   xw_hbm.at[horv.at[pl.ds(i * CHUNK, CHUNK)]],
                   rows.at[pl.ds(i * CHUNK, CHUNK), :], semg)
               for i in range(SUBS)]
        vds = [pltpu.async_copy(
                   invdeg_hbm.at[verv.at[pl.ds(i * CHUNK, CHUNK)]],
                   valsv.at[pl.ds(i * CHUNK, CHUNK)], semv)
               for i in range(SUBS)]
        return gds, vds

    def consume(bufs, gds, vds, base, validf):
        (srcv, dstv, relv, horv, verv, src2d, valsv, rows,
         _, _, _, sems) = bufs
        for d in vds:
            d.wait()
        vw = pltpu.async_copy(valsv, vals_hbm.at[pl.ds(base, SUP)], semw)
        for d in gds:
            d.wait()
        for q in range(SUP // 16):
            v16 = valsv[pl.ds(q * 16, 16)] * validf
            for l in range(16):
                e = q * 16 + l
                rows[e, :] = rows[e, :] * v16[l]
        sds = [pltpu.async_copy(rows.at[pl.ds(i * CHUNK, CHUNK), :],
                                acc.at[src2d.at[i]], sems, add=True)
               for i in range(SUBS)]
        return vw, sds

    b0, _ = chunk_of(0)
    b1, _ = chunk_of(1)
    fire_idx(b0, P0)
    fire_idx(b1, P1)

    def body(k, _):
        base0, val0 = chunk_of(2 * k)
        base1, val1 = chunk_of(2 * k + 1)
        wait_idx(base0, P0)
        gds0, vds0 = compute_and_fire(P0)
        wait_idx(base1, P1)
        gds1, vds1 = compute_and_fire(P1)

        vw0, sds0 = consume(P0, gds0, vds0, base0, val0)
        vw1, sds1 = consume(P1, gds1, vds1, base1, val1)

        @pl.when(k < _NB - 1)
        def _():
            nb0, _ = chunk_of(2 * k + 2)
            nb1, _ = chunk_of(2 * k + 3)
            fire_idx(nb0, P0)
            fire_idx(nb1, P1)
        vw0.wait()
        vw1.wait()
        for d in sds0:
            d.wait()
        for d in sds1:
            d.wait()
        return ()

    lax.fori_loop(0, _NB, body, (), unroll=False)
    plsc.subcore_barrier()
    pltpu.sync_copy(acc.at[pl.ds(s * rows_per_tile, rows_per_tile), :], bounce)
    pltpu.sync_copy(bounce,
                    out_hbm.at[c, pl.ds(s * rows_per_tile, rows_per_tile), :])


@functools.lru_cache(maxsize=None)
def _sc_edge2():
    return pl.kernel(
        _sc_edge2_body,
        compiler_params=pltpu.CompilerParams(use_tc_tiling_on_sc=False),
        out_type=jax.ShapeDtypeStruct((NC, NPAD, H), jnp.float32),
        mesh=_mesh(),
        scratch_types=(
            _edge_bufs() + _edge_bufs() + [
                pltpu.VMEM((NPAD // NS, H), jnp.float32),
                pltpu.VMEM_SHARED((NPAD, H), jnp.float32),
            ] + [pltpu.SemaphoreType.DMA] * 6
        ),
    )


def _sc_edge2_body(src_hbm, dst_hbm, rel_hbm, tab_hbm, vals_hbm, zeros_hbm,
                   out_hbm,
                   srcv0, dstv0, relv0, horv0, verv0, src2d0, valsv0, rows0,
                   srcv1, dstv1, relv1, horv1, verv1, src2d1, valsv1, rows1,
                   bounce, acc,
                   semi0, semi1, semg0, semg1, sems0, sems1):
    c = lax.axis_index("c")
    s = lax.axis_index("s")
    wid = _wid()
    rows_per_tile = NPAD // NS
    pltpu.sync_copy(zeros_hbm.at[pl.ds(s * rows_per_tile, rows_per_tile), :],
                    bounce)
    pltpu.sync_copy(bounce, acc.at[pl.ds(s * rows_per_tile, rows_per_tile), :])
    plsc.subcore_barrier()

    P0 = (srcv0, dstv0, relv0, horv0, src2d0, valsv0, rows0,
          semi0, semg0, sems0)
    P1 = (srcv1, dstv1, relv1, horv1, src2d1, valsv1, rows1,
          semi1, semg1, sems1)

    def chunk_of(j):
        cidr = j * NW + wid
        cid = jnp.minimum(cidr, NSUP - 1)
        return cid * SUP, jnp.where(cidr < NSUP, 1.0, 0.0).astype(jnp.float32)

    def fire_idx(base, bufs):
        (srcv, dstv, relv, _, _, valsv, _, semi, _, _) = bufs
        pltpu.async_copy(src_hbm.at[pl.ds(base, SUP)], srcv, semi)
        pltpu.async_copy(dst_hbm.at[pl.ds(base, SUP)], dstv, semi)
        pltpu.async_copy(rel_hbm.at[pl.ds(base, SUP)], relv, semi)
        pltpu.async_copy(vals_hbm.at[pl.ds(base, SUP)], valsv, semi)

    def wait_idx(base, bufs):
        (srcv, dstv, relv, _, _, valsv, _, semi, _, _) = bufs
        pltpu.make_async_copy(src_hbm.at[pl.ds(base, SUP)], srcv, semi).wait()
        pltpu.make_async_copy(dst_hbm.at[pl.ds(base, SUP)], dstv, semi).wait()
        pltpu.make_async_copy(rel_hbm.at[pl.ds(base, SUP)], relv, semi).wait()
        pltpu.make_async_copy(vals_hbm.at[pl.ds(base, SUP)], valsv,
                              semi).wait()

    def compute_and_fire(bufs):
        (srcv, dstv, relv, horv, src2d, _, rows, _, semg, _) = bufs
        for q in range(SUP // 16):
            sl = pl.ds(q * 16, 16)
            horv[sl] = dstv[sl] * R + relv[sl]
            src2d[q // 8, pl.ds((q % 8) * 16, 16)] = srcv[sl]
        return [pltpu.async_copy(
                    tab_hbm.at[horv.at[pl.ds(i * CHUNK, CHUNK)]],
                    rows.at[pl.ds(i * CHUNK, CHUNK), :], semg)
                for i in range(SUBS)]

    def consume(bufs, gds, validf):
        (_, _, _, _, src2d, valsv, rows, _, _, sems) = bufs
        for d in gds:
            d.wait()
        for q in range(SUP // 16):
            v16 = valsv[pl.ds(q * 16, 16)] * validf
            for l in range(16):
                e = q * 16 + l
                rows[e, :] = rows[e, :] * v16[l]
        return [pltpu.async_copy(rows.at[pl.ds(i * CHUNK, CHUNK), :],
                                 acc.at[src2d.at[i]], sems, add=True)
                for i in range(SUBS)]

    b0, _ = chunk_of(0)
    b1, _ = chunk_of(1)
    fire_idx(b0, P0)
    fire_idx(b1, P1)

    def body(k, _):
        base0, val0 = chunk_of(2 * k)
        base1, val1 = chunk_of(2 * k + 1)
        wait_idx(base0, P0)
        gds0 = compute_and_fire(P0)
        wait_idx(base1, P1)
        gds1 = compute_and_fire(P1)

        sds0 = consume(P0, gds0, val0)
        sds1 = consume(P1, gds1, val1)

        @pl.when(k < _NB - 1)
        def _():
            nb0, _ = chunk_of(2 * k + 2)
            nb1, _ = chunk_of(2 * k + 3)
            fire_idx(nb0, P0)
            fire_idx(nb1, P1)
        for d in sds0:
            d.wait()
        for d in sds1:
            d.wait()
        return ()

    lax.fori_loop(0, _NB, body, (), unroll=False)
    plsc.subcore_barrier()
    pltpu.sync_copy(acc.at[pl.ds(s * rows_per_tile, rows_per_tile), :], bounce)
    pltpu.sync_copy(bounce,
                    out_hbm.at[c, pl.ds(s * rows_per_tile, rows_per_tile), :])


_BN = 2000
_GB = N // _BN


def _tc_xw_body(emb_ref, w1_ref, xw_ref):
    xw_ref[...] = jnp.dot(emb_ref[...], w1_ref[...],
                          preferred_element_type=jnp.float32)


def _tc_xw(emb, w1r):
    return pl.pallas_call(
        _tc_xw_body,
        grid=(_GB,),
        in_specs=[
            pl.BlockSpec((_BN, EMB), lambda i: (i, 0)),
            pl.BlockSpec((EMB, R * H), lambda i: (0, 0)),
        ],
        out_specs=pl.BlockSpec((_BN, R * H), lambda i: (i, 0)),
        out_shape=jax.ShapeDtypeStruct((N, R * H), jnp.float32),
    )(emb, w1r)


def _tc_invdeg_body(h0_ref, h1_ref, invd_ref):
    invd_ref[...] = 1.0 / (h0_ref[...] + h1_ref[...])


def _tc_invdeg(h0, h1):
    hr = (N * R) // 128
    return pl.pallas_call(
        _tc_invdeg_body,
        out_shape=jax.ShapeDtypeStruct((hr, 128), jnp.float32),
    )(h0, h1)


def _tc_t2_body(p0_ref, p1_ref, b1_ref, w2a_ref, w2b_ref, ta_ref, tb_ref):
    h = jnp.maximum(p0_ref[0] + p1_ref[0] + b1_ref[...], 0.0)
    ta_ref[...] = jnp.dot(h, w2a_ref[...], preferred_element_type=jnp.float32)
    tb_ref[...] = jnp.dot(h, w2b_ref[...], preferred_element_type=jnp.float32)


def _tc_t2(p, b1, w2ra, w2rb):
    return pl.pallas_call(
        _tc_t2_body,
        grid=(_GB,),
        in_specs=[
            pl.BlockSpec((1, _BN, H), lambda i: (0, i, 0)),
            pl.BlockSpec((1, _BN, H), lambda i: (1, i, 0)),
            pl.BlockSpec((1, H), lambda i: (0, 0)),
            pl.BlockSpec((H, R * H), lambda i: (0, 0)),
            pl.BlockSpec((H, R * H), lambda i: (0, 0)),
        ],
        out_specs=[
            pl.BlockSpec((_BN, R * H), lambda i: (i, 0)),
            pl.BlockSpec((_BN, R * H), lambda i: (i, 0)),
        ],
        out_shape=[
            jax.ShapeDtypeStruct((N, R * H), jnp.float32),
            jax.ShapeDtypeStruct((N, R * H), jnp.float32),
        ],
    )(p, p, b1, w2ra, w2rb)


def _tc_final_body(a0_ref, a1_ref, b0_ref, b1_ref, b2_ref, o_ref):
    lo = a0_ref[0] + a1_ref[0]
    hi = b0_ref[0] + b1_ref[0]
    o_ref[...] = jnp.concatenate([lo, hi], axis=1) + b2_ref[...]


def _tc_final(pa, pb, b2):
    return pl.pallas_call(
        _tc_final_body,
        grid=(_GB,),
        in_specs=[
            pl.BlockSpec((1, _BN, H), lambda i: (0, i, 0)),
            pl.BlockSpec((1, _BN, H), lambda i: (1, i, 0)),
            pl.BlockSpec((1, _BN, H), lambda i: (0, i, 0)),
            pl.BlockSpec((1, _BN, H), lambda i: (1, i, 0)),
            pl.BlockSpec((1, C), lambda i: (0, 0)),
        ],
        out_specs=pl.BlockSpec((_BN, C), lambda i: (i, 0)),
        out_shape=jax.ShapeDtypeStruct((N, C), jnp.float32),
    )(pa, pa, pb, pb, b2)


def kernel(src, dst, rel, embeddings, weights1, weights2, bias1, bias2):
    w1r = weights1.transpose(1, 0, 2).reshape(EMB, R * H)
    w2ra = weights2[:, :, :16].transpose(1, 0, 2).reshape(H, R * H)
    w2rb = weights2[:, :, 16:].transpose(1, 0, 2).reshape(H, R * H)

    zeros_hist = jnp.zeros((N * R,), jnp.float32)
    zeros16 = jnp.zeros((NPAD, H), jnp.float32)

    hist = _sc_hist()(src, rel, zeros_hist)
    h0 = hist[:N * R].reshape((N * R) // 128, 128)
    h1 = hist[N * R:].reshape((N * R) // 128, 128)

    xw = _tc_xw(embeddings, w1r).reshape(N * R, H)
    invdeg = _tc_invdeg(h0, h1).reshape(N * R)

    vals, p1 = _sc_edge16()(src, dst, rel, xw, invdeg, zeros16)

    ta, tb = _tc_t2(p1, bias1.reshape(1, H), w2ra, w2rb)
    pa = _sc_edge2()(src, dst, rel, ta.reshape(N * R, H), vals, zeros16)
    pb = _sc_edge2()(src, dst, rel, tb.reshape(N * R, H), vals, zeros16)

    return _tc_final(pa, pb, bias2.reshape(1, C))


# speedup vs baseline: 1.0120x; 1.0120x over previous
import functools

import jax
import jax.numpy as jnp
from jax import lax
from jax.experimental import pallas as pl
from jax.experimental.pallas import tpu as pltpu
from jax.experimental.pallas import tpu_sc as plsc

N = 50000
R = 8
E = 800000
EMB = 128
H = 16
C = 32

NC = 2
NS = 16
NW = NC * NS
CHUNK = 320
SUBS = 2
SUP = CHUNK * SUBS
NSUP = E // SUP
ITERS = (NSUP + NW - 1) // NW
NPAD = 50048


@functools.lru_cache(maxsize=None)
def _mesh():
    return plsc.VectorSubcoreMesh(core_axis_name="c", subcore_axis_name="s",
                                  num_cores=NC, num_subcores=NS)


def _wid():
    return lax.axis_index("s") * NC + lax.axis_index("c")


@functools.lru_cache(maxsize=None)
def _sc_hist():
    return pl.kernel(
        _sc_hist_body,
        compiler_params=pltpu.CompilerParams(use_tc_tiling_on_sc=False),
        out_type=jax.ShapeDtypeStruct((NC * N * R,), jnp.float32),
        mesh=_mesh(),
        scratch_types=[
            pltpu.VMEM((SUP,), jnp.int32),
            pltpu.VMEM((SUP,), jnp.int32),
            pltpu.VMEM((SUBS, CHUNK), jnp.int32),
            pltpu.VMEM((SUP,), jnp.float32),
            pltpu.VMEM(((N * R) // NS,), jnp.float32),
            pltpu.VMEM_SHARED((N * R,), jnp.float32),
            pltpu.SemaphoreType.DMA,
            pltpu.SemaphoreType.DMA,
        ],
    )


def _sc_hist_body(src_hbm, rel_hbm, zeros_hbm, out_hbm, srcv, relv, ver2d,
                  onesv, bounce, acc, semi, sems):
    c = lax.axis_index("c")
    s = lax.axis_index("s")
    wid = _wid()
    words = (N * R) // NS
    pltpu.sync_copy(zeros_hbm.at[pl.ds(s * words, words)], bounce)
    pltpu.sync_copy(bounce, acc.at[pl.ds(s * words, words)])
    ones16 = jnp.full((16,), 1.0, dtype=jnp.float32)
    for j in range(SUP // 16):
        onesv[pl.ds(j * 16, 16)] = ones16
    plsc.subcore_barrier()

    def body(g, _):
        cid = g * NW + wid

        @pl.when(cid < NSUP)
        def _():
            base = cid * SUP
            dls = [pltpu.async_copy(src_hbm.at[pl.ds(base, SUP)], srcv, semi),
                   pltpu.async_copy(rel_hbm.at[pl.ds(base, SUP)], relv, semi)]
            for d in dls:
                d.wait()
            for q in range(SUP // 16):
                sl = pl.ds(q * 16, 16)
                ver2d[q // 20, pl.ds((q % 20) * 16, 16)] = srcv[sl] * R + relv[sl]
            sds = [pltpu.async_copy(onesv.at[pl.ds(i * CHUNK, CHUNK)],
                                    acc.at[ver2d.at[i]], sems, add=True)
                   for i in range(SUBS)]
            for d in sds:
                d.wait()

        return ()

    lax.fori_loop(0, ITERS, body, (), unroll=False)
    plsc.subcore_barrier()
    pltpu.sync_copy(acc.at[pl.ds(s * words, words)], bounce)
    pltpu.sync_copy(bounce, out_hbm.at[pl.ds(c * (N * R) + s * words, words)])


_NB = ITERS // 2


def _edge_bufs():
    return [
        pltpu.VMEM((SUP,), jnp.int32),
        pltpu.VMEM((SUP,), jnp.int32),
        pltpu.VMEM((SUP,), jnp.int32),
        pltpu.VMEM((SUP,), jnp.int32),
        pltpu.VMEM((SUP,), jnp.int32),
        pltpu.VMEM((SUBS, CHUNK), jnp.int32),
        pltpu.VMEM((SUP,), jnp.float32),
        pltpu.VMEM((SUP, H), jnp.float32),
    ]


@functools.lru_cache(maxsize=None)
def _sc_edge16():
    return pl.kernel(
        _sc_edge16_body,
        compiler_params=pltpu.CompilerParams(use_tc_tiling_on_sc=False),
        out_type=(
            jax.ShapeDtypeStruct((E,), jnp.float32),
            jax.ShapeDtypeStruct((NC, NPAD, H), jnp.float32),
        ),
        mesh=_mesh(),
        scratch_types=(
            _edge_bufs() + _edge_bufs() + [
                pltpu.VMEM((NPAD // NS, H), jnp.float32),
                pltpu.VMEM_SHARED((NPAD, H), jnp.float32),
            ] + [pltpu.SemaphoreType.DMA] * 9
        ),
    )


def _sc_edge16_body(src_hbm, dst_hbm, rel_hbm, xw_hbm, invdeg_hbm, zeros_hbm,
                    vals_hbm, out_hbm,
                    srcv0, dstv0, relv0, horv0, verv0, src2d0, valsv0, rows0,
                    srcv1, dstv1, relv1, horv1, verv1, src2d1, valsv1, rows1,
                    bounce, acc,
                    semi0, semi1, semg0, semg1, semv0, semv1, sems0, sems1,
                    semw):
    c = lax.axis_index("c")
    s = lax.axis_index("s")
    wid = _wid()
    rows_per_tile = NPAD // NS
    pltpu.sync_copy(zeros_hbm.at[pl.ds(s * rows_per_tile, rows_per_tile), :],
                    bounce)
    pltpu.sync_copy(bounce, acc.at[pl.ds(s * rows_per_tile, rows_per_tile), :])
    plsc.subcore_barrier()

    P0 = (srcv0, dstv0, relv0, horv0, verv0, src2d0, valsv0, rows0,
          semi0, semg0, semv0, sems0)
    P1 = (srcv1, dstv1, relv1, horv1, verv1, src2d1, valsv1, rows1,
          semi1, semg1, semv1, sems1)

    def chunk_of(j):
        cidr = j * NW + wid
        cid = jnp.minimum(cidr, NSUP - 1)
        return cid * SUP, jnp.where(cidr < NSUP, 1.0, 0.0).astype(jnp.float32)

    def fire_idx(base, bufs):
        (srcv, dstv, relv, _, _, _, _, _, semi, _, _, _) = bufs
        pltpu.async_copy(src_hbm.at[pl.ds(base, SUP)], srcv, semi)
        pltpu.async_copy(dst_hbm.at[pl.ds(base, SUP)], dstv, semi)
        pltpu.async_copy(rel_hbm.at[pl.ds(base, SUP)], relv, semi)

    def wait_idx(base, bufs):
        (srcv, dstv, relv, _, _, _, _, _, semi, _, _, _) = bufs
        pltpu.make_async_copy(src_hbm.at[pl.ds(base, SUP)], srcv, semi).wait()
        pltpu.make_async_copy(dst_hbm.at[pl.ds(base, SUP)], dstv, semi).wait()
        pltpu.make_async_copy(rel_hbm.at[pl.ds(base, SUP)], relv, semi).wait()

    def compute_and_fire(bufs):
        (srcv, dstv, relv, horv, verv, src2d, valsv, rows,
         _, semg, semv, _) = bufs
        for q in range(SUP // 16):
            sl = pl.ds(q * 16, 16)
            rj = relv[sl]
            sj = srcv[sl]
            horv[sl] = dstv[sl] * R + rj
            verv[sl] = sj * R + rj
            src2d[q // 20, pl.ds((q % 20) * 16, 16)] = sj
        gds = [pltpu.async_copy(
                   xw_hbm.at[horv.at[pl.ds(i * CHUNK, CHUNK)]],
                   rows.at[pl.ds(i * CHUNK, CHUNK), :], semg)
               for i in range(SUBS)]
        vds = [pltpu.async_copy(
                   invdeg_hbm.at[verv.at[pl.ds(i * CHUNK, CHUNK)]],
                   valsv.at[pl.ds(i * CHUNK, CHUNK)], semv)
               for i in range(SUBS)]
        return gds, vds

    def consume(bufs, gds, vds, base, validf):
        (srcv, dstv, relv, horv, verv, src2d, valsv, rows,
         _, _, _, sems) = bufs
        for d in vds:
            d.wait()
        vw = pltpu.async_copy(valsv, vals_hbm.at[pl.ds(base, SUP)], semw)
        for d in gds:
            d.wait()
        for q in range(SUP // 16):
            v16 = valsv[pl.ds(q * 16, 16)] * validf
            for l in range(16):
                e = q * 16 + l
                rows[e, :] = rows[e, :] * v16[l]
        sds = [pltpu.async_copy(rows.at[pl.ds(i * CHUNK, CHUNK), :],
                                acc.at[src2d.at[i]], sems, add=True)
               for i in range(SUBS)]
        return vw, sds

    b0, _ = chunk_of(0)
    b1, _ = chunk_of(1)
    fire_idx(b0, P0)
    fire_idx(b1, P1)

    def body(k, _):
        base0, val0 = chunk_of(2 * k)
        base1, val1 = chunk_of(2 * k + 1)
        wait_idx(base0, P0)
        gds0, vds0 = compute_and_fire(P0)
        wait_idx(base1, P1)
        gds1, vds1 = compute_and_fire(P1)

        vw0, sds0 = consume(P0, gds0, vds0, base0, val0)
        vw1, sds1 = consume(P1, gds1, vds1, base1, val1)

        @pl.when(k < _NB - 1)
        def _():
            nb0, _ = chunk_of(2 * k + 2)
            nb1, _ = chunk_of(2 * k + 3)
            fire_idx(nb0, P0)
            fire_idx(nb1, P1)
        vw0.wait()
        vw1.wait()
        for d in sds0:
            d.wait()
        for d in sds1:
            d.wait()
        return ()

    lax.fori_loop(0, _NB, body, (), unroll=False)
    plsc.subcore_barrier()
    pltpu.sync_copy(acc.at[pl.ds(s * rows_per_tile, rows_per_tile), :], bounce)
    pltpu.sync_copy(bounce,
                    out_hbm.at[c, pl.ds(s * rows_per_tile, rows_per_tile), :])


@functools.lru_cache(maxsize=None)
def _sc_edge2():
    return pl.kernel(
        _sc_edge2_body,
        compiler_params=pltpu.CompilerParams(use_tc_tiling_on_sc=False),
        out_type=jax.ShapeDtypeStruct((NC, NPAD, H), jnp.float32),
        mesh=_mesh(),
        scratch_types=(
            _edge_bufs() + _edge_bufs() + [
                pltpu.VMEM((NPAD // NS, H), jnp.float32),
                pltpu.VMEM_SHARED((NPAD, H), jnp.float32),
            ] + [pltpu.SemaphoreType.DMA] * 6
        ),
    )


def _sc_edge2_body(src_hbm, dst_hbm, rel_hbm, tab_hbm, vals_hbm, zeros_hbm,
                   out_hbm,
                   srcv0, dstv0, relv0, horv0, verv0, src2d0, valsv0, rows0,
                   srcv1, dstv1, relv1, horv1, verv1, src2d1, valsv1, rows1,
                   bounce, acc,
                   semi0, semi1, semg0, semg1, sems0, sems1):
    c = lax.axis_index("c")
    s = lax.axis_index("s")
    wid = _wid()
    rows_per_tile = NPAD // NS
    pltpu.sync_copy(zeros_hbm.at[pl.ds(s * rows_per_tile, rows_per_tile), :],
                    bounce)
    pltpu.sync_copy(bounce, acc.at[pl.ds(s * rows_per_tile, rows_per_tile), :])
    plsc.subcore_barrier()

    P0 = (srcv0, dstv0, relv0, horv0, src2d0, valsv0, rows0,
          semi0, semg0, sems0)
    P1 = (srcv1, dstv1, relv1, horv1, src2d1, valsv1, rows1,
          semi1, semg1, sems1)

    def chunk_of(j):
        cidr = j * NW + wid
        cid = jnp.minimum(cidr, NSUP - 1)
        return cid * SUP, jnp.where(cidr < NSUP, 1.0, 0.0).astype(jnp.float32)

    def fire_idx(base, bufs):
        (srcv, dstv, relv, _, _, valsv, _, semi, _, _) = bufs
        pltpu.async_copy(src_hbm.at[pl.ds(base, SUP)], srcv, semi)
        pltpu.async_copy(dst_hbm.at[pl.ds(base, SUP)], dstv, semi)
        pltpu.async_copy(rel_hbm.at[pl.ds(base, SUP)], relv, semi)
        pltpu.async_copy(vals_hbm.at[pl.ds(base, SUP)], valsv, semi)

    def wait_idx(base, bufs):
        (srcv, dstv, relv, _, _, valsv, _, semi, _, _) = bufs
        pltpu.make_async_copy(src_hbm.at[pl.ds(base, SUP)], srcv, semi).wait()
        pltpu.make_async_copy(dst_hbm.at[pl.ds(base, SUP)], dstv, semi).wait()
        pltpu.make_async_copy(rel_hbm.at[pl.ds(base, SUP)], relv, semi).wait()
        pltpu.make_async_copy(vals_hbm.at[pl.ds(base, SUP)], valsv,
                              semi).wait()

    def compute_and_fire(bufs):
        (srcv, dstv, relv, horv, src2d, _, rows, _, semg, _) = bufs
        for q in range(SUP // 16):
            sl = pl.ds(q * 16, 16)
            horv[sl] = dstv[sl] * R + relv[sl]
            src2d[q // 20, pl.ds((q % 20) * 16, 16)] = srcv[sl]
        return [pltpu.async_copy(
                    tab_hbm.at[horv.at[pl.ds(i * CHUNK, CHUNK)]],
                    rows.at[pl.ds(i * CHUNK, CHUNK), :], semg)
                for i in range(SUBS)]

    def consume(bufs, gds, validf):
        (_, _, _, _, src2d, valsv, rows, _, _, sems) = bufs
        for d in gds:
            d.wait()
        for q in range(SUP // 16):
            v16 = valsv[pl.ds(q * 16, 16)] * validf
            for l in range(16):
                e = q * 16 + l
                rows[e, :] = rows[e, :] * v16[l]
        return [pltpu.async_copy(rows.at[pl.ds(i * CHUNK, CHUNK), :],
                                 acc.at[src2d.at[i]], sems, add=True)
                for i in range(SUBS)]

    b0, _ = chunk_of(0)
    b1, _ = chunk_of(1)
    fire_idx(b0, P0)
    fire_idx(b1, P1)

    def body(k, _):
        base0, val0 = chunk_of(2 * k)
        base1, val1 = chunk_of(2 * k + 1)
        wait_idx(base0, P0)
        gds0 = compute_and_fire(P0)
        wait_idx(base1, P1)
        gds1 = compute_and_fire(P1)

        sds0 = consume(P0, gds0, val0)
        sds1 = consume(P1, gds1, val1)

        @pl.when(k < _NB - 1)
        def _():
            nb0, _ = chunk_of(2 * k + 2)
            nb1, _ = chunk_of(2 * k + 3)
            fire_idx(nb0, P0)
            fire_idx(nb1, P1)
        for d in sds0:
            d.wait()
        for d in sds1:
            d.wait()
        return ()

    lax.fori_loop(0, _NB, body, (), unroll=False)
    plsc.subcore_barrier()
    pltpu.sync_copy(acc.at[pl.ds(s * rows_per_tile, rows_per_tile), :], bounce)
    pltpu.sync_copy(bounce,
                    out_hbm.at[c, pl.ds(s * rows_per_tile, rows_per_tile), :])


_BN = 2000
_GB = N // _BN


def _tc_xw_body(emb_ref, w1_ref, xw_ref):
    xw_ref[...] = jnp.dot(emb_ref[...], w1_ref[...],
                          preferred_element_type=jnp.float32)


def _tc_xw(emb, w1r):
    return pl.pallas_call(
        _tc_xw_body,
        grid=(_GB,),
        in_specs=[
            pl.BlockSpec((_BN, EMB), lambda i: (i, 0)),
            pl.BlockSpec((EMB, R * H), lambda i: (0, 0)),
        ],
        out_specs=pl.BlockSpec((_BN, R * H), lambda i: (i, 0)),
        out_shape=jax.ShapeDtypeStruct((N, R * H), jnp.float32),
    )(emb, w1r)


def _tc_invdeg_body(h0_ref, h1_ref, invd_ref):
    invd_ref[...] = 1.0 / (h0_ref[...] + h1_ref[...])


def _tc_invdeg(h0, h1):
    hr = (N * R) // 128
    return pl.pallas_call(
        _tc_invdeg_body,
        out_shape=jax.ShapeDtypeStruct((hr, 128), jnp.float32),
    )(h0, h1)


def _tc_t2_body(p0_ref, p1_ref, b1_ref, w2a_ref, w2b_ref, ta_ref, tb_ref):
    h = jnp.maximum(p0_ref[0] + p1_ref[0] + b1_ref[...], 0.0)
    ta_ref[...] = jnp.dot(h, w2a_ref[...], preferred_element_type=jnp.float32)
    tb_ref[...] = jnp.dot(h, w2b_ref[...], preferred_element_type=jnp.float32)


def _tc_t2(p, b1, w2ra, w2rb):
    return pl.pallas_call(
        _tc_t2_body,
        grid=(_GB,),
        in_specs=[
            pl.BlockSpec((1, _BN, H), lambda i: (0, i, 0)),
            pl.BlockSpec((1, _BN, H), lambda i: (1, i, 0)),
            pl.BlockSpec((1, H), lambda i: (0, 0)),
            pl.BlockSpec((H, R * H), lambda i: (0, 0)),
            pl.BlockSpec((H, R * H), lambda i: (0, 0)),
        ],
        out_specs=[
            pl.BlockSpec((_BN, R * H), lambda i: (i, 0)),
            pl.BlockSpec((_BN, R * H), lambda i: (i, 0)),
        ],
        out_shape=[
            jax.ShapeDtypeStruct((N, R * H), jnp.float32),
            jax.ShapeDtypeStruct((N, R * H), jnp.float32),
        ],
    )(p, p, b1, w2ra, w2rb)


def _tc_final_body(a0_ref, a1_ref, b0_ref, b1_ref, b2_ref, o_ref):
    lo = a0_ref[0] + a1_ref[0]
    hi = b0_ref[0] + b1_ref[0]
    o_ref[...] = jnp.concatenate([lo, hi], axis=1) + b2_ref[...]


def _tc_final(pa, pb, b2):
    return pl.pallas_call(
        _tc_final_body,
        grid=(_GB,),
        in_specs=[
            pl.BlockSpec((1, _BN, H), lambda i: (0, i, 0)),
            pl.BlockSpec((1, _BN, H), lambda i: (1, i, 0)),
            pl.BlockSpec((1, _BN, H), lambda i: (0, i, 0)),
            pl.BlockSpec((1, _BN, H), lambda i: (1, i, 0)),
            pl.BlockSpec((1, C), lambda i: (0, 0)),
        ],
        out_specs=pl.BlockSpec((_BN, C), lambda i: (i, 0)),
        out_shape=jax.ShapeDtypeStruct((N, C), jnp.float32),
    )(pa, pa, pb, pb, b2)


def kernel(src, dst, rel, embeddings, weights1, weights2, bias1, bias2):
    w1r = weights1.transpose(1, 0, 2).reshape(EMB, R * H)
    w2ra = weights2[:, :, :16].transpose(1, 0, 2).reshape(H, R * H)
    w2rb = weights2[:, :, 16:].transpose(1, 0, 2).reshape(H, R * H)

    zeros_hist = jnp.zeros((N * R,), jnp.float32)
    zeros16 = jnp.zeros((NPAD, H), jnp.float32)

    hist = _sc_hist()(src, rel, zeros_hist)
    h0 = hist[:N * R].reshape((N * R) // 128, 128)
    h1 = hist[N * R:].reshape((N * R) // 128, 128)

    xw = _tc_xw(embeddings, w1r).reshape(N * R, H)
    invdeg = _tc_invdeg(h0, h1).reshape(N * R)

    vals, p1 = _sc_edge16()(src, dst, rel, xw, invdeg, zeros16)

    ta, tb = _tc_t2(p1, bias1.reshape(1, H), w2ra, w2rb)
    pa = _sc_edge2()(src, dst, rel, ta.reshape(N * R, H), vals, zeros16)
    pb = _sc_edge2()(src, dst, rel, tb.reshape(N * R, H), vals, zeros16)

    return _tc_final(pa, pb, bias2.reshape(1, C))


# speedup vs baseline: 1.0260x; 1.0139x over previous
import functools

import jax
import jax.numpy as jnp
from jax import lax
from jax.experimental import pallas as pl
from jax.experimental.pallas import tpu as pltpu
from jax.experimental.pallas import tpu_sc as plsc

N = 50000
R = 8
E = 800000
EMB = 128
H = 16
C = 32

NC = 2
NS = 16
NW = NC * NS
CHUNK = 640
SUBS = 1
SUP = CHUNK * SUBS
NSUP = E // SUP
ITERS = (NSUP + NW - 1) // NW
NPAD = 50048


@functools.lru_cache(maxsize=None)
def _mesh():
    return plsc.VectorSubcoreMesh(core_axis_name="c", subcore_axis_name="s",
                                  num_cores=NC, num_subcores=NS)


def _wid():
    return lax.axis_index("s") * NC + lax.axis_index("c")


@functools.lru_cache(maxsize=None)
def _sc_hist():
    return pl.kernel(
        _sc_hist_body,
        compiler_params=pltpu.CompilerParams(use_tc_tiling_on_sc=False),
        out_type=jax.ShapeDtypeStruct((NC * N * R,), jnp.float32),
        mesh=_mesh(),
        scratch_types=[
            pltpu.VMEM((SUP,), jnp.int32),
            pltpu.VMEM((SUP,), jnp.int32),
            pltpu.VMEM((SUBS, CHUNK), jnp.int32),
            pltpu.VMEM((SUP,), jnp.float32),
            pltpu.VMEM(((N * R) // NS,), jnp.float32),
            pltpu.VMEM_SHARED((N * R,), jnp.float32),
            pltpu.SemaphoreType.DMA,
            pltpu.SemaphoreType.DMA,
        ],
    )


def _sc_hist_body(src_hbm, rel_hbm, zeros_hbm, out_hbm, srcv, relv, ver2d,
                  onesv, bounce, acc, semi, sems):
    c = lax.axis_index("c")
    s = lax.axis_index("s")
    wid = _wid()
    words = (N * R) // NS
    pltpu.sync_copy(zeros_hbm.at[pl.ds(s * words, words)], bounce)
    pltpu.sync_copy(bounce, acc.at[pl.ds(s * words, words)])
    ones16 = jnp.full((16,), 1.0, dtype=jnp.float32)
    for j in range(SUP // 16):
        onesv[pl.ds(j * 16, 16)] = ones16
    plsc.subcore_barrier()

    def body(g, _):
        cid = g * NW + wid

        @pl.when(cid < NSUP)
        def _():
            base = cid * SUP
            dls = [pltpu.async_copy(src_hbm.at[pl.ds(base, SUP)], srcv, semi),
                   pltpu.async_copy(rel_hbm.at[pl.ds(base, SUP)], relv, semi)]
            for d in dls:
                d.wait()
            for q in range(SUP // 16):
                sl = pl.ds(q * 16, 16)
                ver2d[q // 40, pl.ds((q % 40) * 16, 16)] = srcv[sl] * R + relv[sl]
            sds = [pltpu.async_copy(onesv.at[pl.ds(i * CHUNK, CHUNK)],
                                    acc.at[ver2d.at[i]], sems, add=True)
                   for i in range(SUBS)]
            for d in sds:
                d.wait()

        return ()

    lax.fori_loop(0, ITERS, body, (), unroll=False)
    plsc.subcore_barrier()
    pltpu.sync_copy(acc.at[pl.ds(s * words, words)], bounce)
    pltpu.sync_copy(bounce, out_hbm.at[pl.ds(c * (N * R) + s * words, words)])


_NB = ITERS // 2


def _edge_bufs():
    return [
        pltpu.VMEM((SUP,), jnp.int32),
        pltpu.VMEM((SUP,), jnp.int32),
        pltpu.VMEM((SUP,), jnp.int32),
        pltpu.VMEM((SUP,), jnp.int32),
        pltpu.VMEM((SUP,), jnp.int32),
        pltpu.VMEM((SUBS, CHUNK), jnp.int32),
        pltpu.VMEM((SUP,), jnp.float32),
        pltpu.VMEM((SUP, H), jnp.float32),
    ]


@functools.lru_cache(maxsize=None)
def _sc_edge16():
    return pl.kernel(
        _sc_edge16_body,
        compiler_params=pltpu.CompilerParams(use_tc_tiling_on_sc=False),
        out_type=(
            jax.ShapeDtypeStruct((E,), jnp.float32),
            jax.ShapeDtypeStruct((NC, NPAD, H), jnp.float32),
        ),
        mesh=_mesh(),
        scratch_types=(
            _edge_bufs() + _edge_bufs() + [
                pltpu.VMEM((NPAD // NS, H), jnp.float32),
                pltpu.VMEM_SHARED((NPAD, H), jnp.float32),
            ] + [pltpu.SemaphoreType.DMA] * 9
        ),
    )


def _sc_edge16_body(src_hbm, dst_hbm, rel_hbm, xw_hbm, invdeg_hbm, zeros_hbm,
                    vals_hbm, out_hbm,
                    srcv0, dstv0, relv0, horv0, verv0, src2d0, valsv0, rows0,
                    srcv1, dstv1, relv1, horv1, verv1, src2d1, valsv1, rows1,
                    bounce, acc,
                    semi0, semi1, semg0, semg1, semv0, semv1, sems0, sems1,
                    semw):
    c = lax.axis_index("c")
    s = lax.axis_index("s")
    wid = _wid()
    rows_per_tile = NPAD // NS
    pltpu.sync_copy(zeros_hbm.at[pl.ds(s * rows_per_tile, rows_per_tile), :],
                    bounce)
    pltpu.sync_copy(bounce, acc.at[pl.ds(s * rows_per_tile, rows_per_tile), :])
    plsc.subcore_barrier()

    P0 = (srcv0, dstv0, relv0, horv0, verv0, src2d0, valsv0, rows0,
          semi0, semg0, semv0, sems0)
    P1 = (srcv1, dstv1, relv1, horv1, verv1, src2d1, valsv1, rows1,
          semi1, semg1, semv1, sems1)

    def chunk_of(j):
        cidr = j * NW + wid
        cid = jnp.minimum(cidr, NSUP - 1)
        return cid * SUP, jnp.where(cidr < NSUP, 1.0, 0.0).astype(jnp.float32)

    def fire_idx(base, bufs):
        (srcv, dstv, relv, _, _, _, _, _, semi, _, _, _) = bufs
        pltpu.async_copy(src_hbm.at[pl.ds(base, SUP)], srcv, semi)
        pltpu.async_copy(dst_hbm.at[pl.ds(base, SUP)], dstv, semi)
        pltpu.async_copy(rel_hbm.at[pl.ds(base, SUP)], relv, semi)

    def wait_idx(base, bufs):
        (srcv, dstv, relv, _, _, _, _, _, semi, _, _, _) = bufs
        pltpu.make_async_copy(src_hbm.at[pl.ds(base, SUP)], srcv, semi).wait()
        pltpu.make_async_copy(dst_hbm.at[pl.ds(base, SUP)], dstv, semi).wait()
        pltpu.make_async_copy(rel_hbm.at[pl.ds(base, SUP)], relv, semi).wait()

    def compute_and_fire(bufs):
        (srcv, dstv, relv, horv, verv, src2d, valsv, rows,
         _, semg, semv, _) = bufs
        for q in range(SUP // 16):
            sl = pl.ds(q * 16, 16)
            rj = relv[sl]
            sj = srcv[sl]
            horv[sl] = dstv[sl] * R + rj
            verv[sl] = sj * R + rj
            src2d[q // 40, pl.ds((q % 40) * 16, 16)] = sj
        gds = [pltpu.async_copy(
                   xw_hbm.at[horv.at[pl.ds(i * CHUNK, CHUNK)]],
                   rows.at[pl.ds(i * CHUNK, CHUNK), :], semg)
               for i in range(SUBS)]
        vds = [pltpu.async_copy(
                   invdeg_hbm.at[verv.at[pl.ds(i * CHUNK, CHUNK)]],
                   valsv.at[pl.ds(i * CHUNK, CHUNK)], semv)
               for i in range(SUBS)]
        return gds, vds

    def consume(bufs, gds, vds, base, validf):
        (srcv, dstv, relv, horv, verv, src2d, valsv, rows,
         _, _, _, sems) = bufs
        for d in vds:
            d.wait()
        vw = pltpu.async_copy(valsv, vals_hbm.at[pl.ds(base, SUP)], semw)
        for d in gds:
            d.wait()
        for q in range(SUP // 16):
            v16 = valsv[pl.ds(q * 16, 16)] * validf
            for l in range(16):
                e = q * 16 + l
                rows[e, :] = rows[e, :] * v16[l]
        sds = [pltpu.async_copy(rows.at[pl.ds(i * CHUNK, CHUNK), :],
                                acc.at[src2d.at[i]], sems, add=True)
               for i in range(SUBS)]
        return vw, sds

    b0, _ = chunk_of(0)
    b1, _ = chunk_of(1)
    fire_idx(b0, P0)
    fire_idx(b1, P1)

    def body(k, _):
        base0, val0 = chunk_of(2 * k)
        base1, val1 = chunk_of(2 * k + 1)
        wait_idx(base0, P0)
        gds0, vds0 = compute_and_fire(P0)
        wait_idx(base1, P1)
        gds1, vds1 = compute_and_fire(P1)

        vw0, sds0 = consume(P0, gds0, vds0, base0, val0)
        vw1, sds1 = consume(P1, gds1, vds1, base1, val1)

        @pl.when(k < _NB - 1)
        def _():
            nb0, _ = chunk_of(2 * k + 2)
            nb1, _ = chunk_of(2 * k + 3)
            fire_idx(nb0, P0)
            fire_idx(nb1, P1)
        vw0.wait()
        vw1.wait()
        for d in sds0:
            d.wait()
        for d in sds1:
            d.wait()
        return ()

    lax.fori_loop(0, _NB, body, (), unroll=False)
    plsc.subcore_barrier()
    pltpu.sync_copy(acc.at[pl.ds(s * rows_per_tile, rows_per_tile), :], bounce)
    pltpu.sync_copy(bounce,
                    out_hbm.at[c, pl.ds(s * rows_per_tile, rows_per_tile), :])


@functools.lru_cache(maxsize=None)
def _sc_edge2():
    return pl.kernel(
        _sc_edge2_body,
        compiler_params=pltpu.CompilerParams(use_tc_tiling_on_sc=False),
        out_type=jax.ShapeDtypeStruct((NC, NPAD, H), jnp.float32),
        mesh=_mesh(),
        scratch_types=(
            _edge_bufs() + _edge_bufs() + [
                pltpu.VMEM((NPAD // NS, H), jnp.float32),
                pltpu.VMEM_SHARED((NPAD, H), jnp.float32),
            ] + [pltpu.SemaphoreType.DMA] * 6
        ),
    )


def _sc_edge2_body(src_hbm, dst_hbm, rel_hbm, tab_hbm, vals_hbm, zeros_hbm,
                   out_hbm,
                   srcv0, dstv0, relv0, horv0, verv0, src2d0, valsv0, rows0,
                   srcv1, dstv1, relv1, horv1, verv1, src2d1, valsv1, rows1,
                   bounce, acc,
                   semi0, semi1, semg0, semg1, sems0, sems1):
    c = lax.axis_index("c")
    s = lax.axis_index("s")
    wid = _wid()
    rows_per_tile = NPAD // NS
    pltpu.sync_copy(zeros_hbm.at[pl.ds(s * rows_per_tile, rows_per_tile), :],
                    bounce)
    pltpu.sync_copy(bounce, acc.at[pl.ds(s * rows_per_tile, rows_per_tile), :])
    plsc.subcore_barrier()

    P0 = (srcv0, dstv0, relv0, horv0, src2d0, valsv0, rows0,
          semi0, semg0, sems0)
    P1 = (srcv1, dstv1, relv1, horv1, src2d1, valsv1, rows1,
          semi1, semg1, sems1)

    def chunk_of(j):
        cidr = j * NW + wid
        cid = jnp.minimum(cidr, NSUP - 1)
        return cid * SUP, jnp.where(cidr < NSUP, 1.0, 0.0).astype(jnp.float32)

    def fire_idx(base, bufs):
        (srcv, dstv, relv, _, _, valsv, _, semi, _, _) = bufs
        pltpu.async_copy(src_hbm.at[pl.ds(base, SUP)], srcv, semi)
        pltpu.async_copy(dst_hbm.at[pl.ds(base, SUP)], dstv, semi)
        pltpu.async_copy(rel_hbm.at[pl.ds(base, SUP)], relv, semi)
        pltpu.async_copy(vals_hbm.at[pl.ds(base, SUP)], valsv, semi)

    def wait_idx(base, bufs):
        (srcv, dstv, relv, _, _, valsv, _, semi, _, _) = bufs
        pltpu.make_async_copy(src_hbm.at[pl.ds(base, SUP)], srcv, semi).wait()
        pltpu.make_async_copy(dst_hbm.at[pl.ds(base, SUP)], dstv, semi).wait()
        pltpu.make_async_copy(rel_hbm.at[pl.ds(base, SUP)], relv, semi).wait()
        pltpu.make_async_copy(vals_hbm.at[pl.ds(base, SUP)], valsv,
                              semi).wait()

    def compute_and_fire(bufs):
        (srcv, dstv, relv, horv, src2d, _, rows, _, semg, _) = bufs
        for q in range(SUP // 16):
            sl = pl.ds(q * 16, 16)
            horv[sl] = dstv[sl] * R + relv[sl]
            src2d[q // 40, pl.ds((q % 40) * 16, 16)] = srcv[sl]
        return [pltpu.async_copy(
                    tab_hbm.at[horv.at[pl.ds(i * CHUNK, CHUNK)]],
                    rows.at[pl.ds(i * CHUNK, CHUNK), :], semg)
                for i in range(SUBS)]

    def consume(bufs, gds, validf):
        (_, _, _, _, src2d, valsv, rows, _, _, sems) = bufs
        for d in gds:
            d.wait()
        for q in range(SUP // 16):
            v16 = valsv[pl.ds(q * 16, 16)] * validf
            for l in range(16):
                e = q * 16 + l
                rows[e, :] = rows[e, :] * v16[l]
        return [pltpu.async_copy(rows.at[pl.ds(i * CHUNK, CHUNK), :],
                                 acc.at[src2d.at[i]], sems, add=True)
                for i in range(SUBS)]

    b0, _ = chunk_of(0)
    b1, _ = chunk_of(1)
    fire_idx(b0, P0)
    fire_idx(b1, P1)

    def body(k, _):
        base0, val0 = chunk_of(2 * k)
        base1, val1 = chunk_of(2 * k + 1)
        wait_idx(base0, P0)
        gds0 = compute_and_fire(P0)
        wait_idx(base1, P1)
        gds1 = compute_and_fire(P1)

        sds0 = consume(P0, gds0, val0)
        sds1 = consume(P1, gds1, val1)

        @pl.when(k < _NB - 1)
        def _():
            nb0, _ = chunk_of(2 * k + 2)
            nb1, _ = chunk_of(2 * k + 3)
            fire_idx(nb0, P0)
            fire_idx(nb1, P1)
        for d in sds0:
            d.wait()
        for d in sds1:
            d.wait()
        return ()

    lax.fori_loop(0, _NB, body, (), unroll=False)
    plsc.subcore_barrier()
    pltpu.sync_copy(acc.at[pl.ds(s * rows_per_tile, rows_per_tile), :], bounce)
    pltpu.sync_copy(bounce,
                    out_hbm.at[c, pl.ds(s * rows_per_tile, rows_per_tile), :])


_BN = 2000
_GB = N // _BN


def _tc_xw_body(emb_ref, w1_ref, xw_ref):
    xw_ref[...] = jnp.dot(emb_ref[...], w1_ref[...],
                          preferred_element_type=jnp.float32)


def _tc_xw(emb, w1r):
    return pl.pallas_call(
        _tc_xw_body,
        grid=(_GB,),
        in_specs=[
            pl.BlockSpec((_BN, EMB), lambda i: (i, 0)),
            pl.BlockSpec((EMB, R * H), lambda i: (0, 0)),
        ],
        out_specs=pl.BlockSpec((_BN, R * H), lambda i: (i, 0)),
        out_shape=jax.ShapeDtypeStruct((N, R * H), jnp.float32),
    )(emb, w1r)


def _tc_invdeg_body(h0_ref, h1_ref, invd_ref):
    invd_ref[...] = 1.0 / (h0_ref[...] + h1_ref[...])


def _tc_invdeg(h0, h1):
    hr = (N * R) // 128
    return pl.pallas_call(
        _tc_invdeg_body,
        out_shape=jax.ShapeDtypeStruct((hr, 128), jnp.float32),
    )(h0, h1)


def _tc_t2_body(p0_ref, p1_ref, b1_ref, w2a_ref, w2b_ref, ta_ref, tb_ref):
    h = jnp.maximum(p0_ref[0] + p1_ref[0] + b1_ref[...], 0.0)
    ta_ref[...] = jnp.dot(h, w2a_ref[...], preferred_element_type=jnp.float32)
    tb_ref[...] = jnp.dot(h, w2b_ref[...], preferred_element_type=jnp.float32)


def _tc_t2(p, b1, w2ra, w2rb):
    return pl.pallas_call(
        _tc_t2_body,
        grid=(_GB,),
        in_specs=[
            pl.BlockSpec((1, _BN, H), lambda i: (0, i, 0)),
            pl.BlockSpec((1, _BN, H), lambda i: (1, i, 0)),
            pl.BlockSpec((1, H), lambda i: (0, 0)),
            pl.BlockSpec((H, R * H), lambda i: (0, 0)),
            pl.BlockSpec((H, R * H), lambda i: (0, 0)),
        ],
        out_specs=[
            pl.BlockSpec((_BN, R * H), lambda i: (i, 0)),
            pl.BlockSpec((_BN, R * H), lambda i: (i, 0)),
        ],
        out_shape=[
            jax.ShapeDtypeStruct((N, R * H), jnp.float32),
            jax.ShapeDtypeStruct((N, R * H), jnp.float32),
        ],
    )(p, p, b1, w2ra, w2rb)


def _tc_final_body(a0_ref, a1_ref, b0_ref, b1_ref, b2_ref, o_ref):
    lo = a0_ref[0] + a1_ref[0]
    hi = b0_ref[0] + b1_ref[0]
    o_ref[...] = jnp.concatenate([lo, hi], axis=1) + b2_ref[...]


def _tc_final(pa, pb, b2):
    return pl.pallas_call(
        _tc_final_body,
        grid=(_GB,),
        in_specs=[
            pl.BlockSpec((1, _BN, H), lambda i: (0, i, 0)),
            pl.BlockSpec((1, _BN, H), lambda i: (1, i, 0)),
            pl.BlockSpec((1, _BN, H), lambda i: (0, i, 0)),
            pl.BlockSpec((1, _BN, H), lambda i: (1, i, 0)),
            pl.BlockSpec((1, C), lambda i: (0, 0)),
        ],
        out_specs=pl.BlockSpec((_BN, C), lambda i: (i, 0)),
        out_shape=jax.ShapeDtypeStruct((N, C), jnp.float32),
    )(pa, pa, pb, pb, b2)


def kernel(src, dst, rel, embeddings, weights1, weights2, bias1, bias2):
    w1r = weights1.transpose(1, 0, 2).reshape(EMB, R * H)
    w2ra = weights2[:, :, :16].transpose(1, 0, 2).reshape(H, R * H)
    w2rb = weights2[:, :, 16:].transpose(1, 0, 2).reshape(H, R * H)

    zeros_hist = jnp.zeros((N * R,), jnp.float32)
    zeros16 = jnp.zeros((NPAD, H), jnp.float32)

    hist = _sc_hist()(src, rel, zeros_hist)
    h0 = hist[:N * R].reshape((N * R) // 128, 128)
    h1 = hist[N * R:].reshape((N * R) // 128, 128)

    xw = _tc_xw(embeddings, w1r).reshape(N * R, H)
    invdeg = _tc_invdeg(h0, h1).reshape(N * R)

    vals, p1 = _sc_edge16()(src, dst, rel, xw, invdeg, zeros16)

    ta, tb = _tc_t2(p1, bias1.reshape(1, H), w2ra, w2rb)
    pa = _sc_edge2()(src, dst, rel, ta.reshape(N * R, H), vals, zeros16)
    pb = _sc_edge2()(src, dst, rel, tb.reshape(N * R, H), vals, zeros16)

    return _tc_final(pa, pb, bias2.reshape(1, C))
